# baseline (speedup 1.0000x reference)
import functools

import jax
import jax.numpy as jnp
from jax import lax
from jax.experimental import pallas as pl
from jax.experimental.pallas import tpu as pltpu
from jax.experimental.pallas import tpu_sc as plsc

NC = 2
NS = 16
L = 16
NW = NC * NS

TOKENS = 8192
D = 768
N_POS = 2048
R = TOKENS // NW
C = 64
NCH = R // C
DL = D // L
DL2 = DL // 2

_mesh = plsc.VectorSubcoreMesh(core_axis_name="c", subcore_axis_name="s")


@functools.partial(
    pl.kernel,
    out_type=(
        jax.ShapeDtypeStruct((TOKENS, D), jnp.float32),
        jax.ShapeDtypeStruct((TOKENS,), jnp.float32),
    ),
    mesh=_mesh,
    compiler_params=pltpu.CompilerParams(needs_layout_passes=False),
    scratch_types=[
        pltpu.VMEM((R,), jnp.int32),
        pltpu.VMEM((R,), jnp.int32),
        pltpu.VMEM((C, D), jnp.float32),
        pltpu.VMEM((C, D // 2), jnp.int32),
        pltpu.VMEM((R,), jnp.float32),
        pltpu.SemaphoreType.DMA,
    ],
)
def _emb_kernel(ids_hbm, pos_hbm, am_hbm, wte_hbm, wpe_hbm, out_hbm, mask_hbm,
                tok_idx, pos_idx, tok_rows, pos_rows, am_v, sem):
    wid = lax.axis_index("s") * NC + lax.axis_index("c")
    base = wid * R

    pltpu.sync_copy(ids_hbm.at[pl.ds(base, R)], tok_idx)
    pltpu.sync_copy(pos_hbm.at[pl.ds(base, R)], pos_idx)

    @pl.loop(0, NCH)
    def _chunk(i):
        sl = pl.ds(i * C, C)
        h1 = pltpu.async_copy(wte_hbm.at[tok_idx.at[sl]], tok_rows, sem)
        h2 = pltpu.async_copy(wpe_hbm.at[pos_idx.at[sl]], pos_rows, sem)

        @pl.when(i == 0)
        def _():
            pltpu.sync_copy(am_hbm.at[pl.ds(base, R)], am_v)

            @pl.loop(0, R // L)
            def _mask(j):
                s = pl.ds(j * L, L)
                am_v[s] = (am_v[s] - 1.0) * 10000.0

            pltpu.sync_copy(am_v, mask_hbm.at[pl.ds(base, R)])

        h1.wait()
        h2.wait()

        @pl.loop(0, C)
        def _row(r):
            for j in range(DL2):
                x = pos_rows[r, pl.ds(j * L, L)]
                lo = plsc.bitcast(lax.shift_left(x, 16), jnp.float32)
                hi = plsc.bitcast(
                    lax.bitwise_and(x, jnp.int32(-65536)), jnp.float32)
                s0 = pl.ds((2 * j) * L, L)
                s1 = pl.ds((2 * j + 1) * L, L)
                tok_rows[r, s0] = tok_rows[r, s0] + lo
                tok_rows[r, s1] = tok_rows[r, s1] + hi

        pltpu.sync_copy(tok_rows, out_hbm.at[pl.ds(base + i * C, C)])


def kernel(input_ids, attention_mask, position_ids, wte, wpe):
    B, S = input_ids.shape
    ids = input_ids.reshape(-1).astype(jnp.int32)
    pos = position_ids.reshape(-1).astype(jnp.int32)
    am = attention_mask.reshape(-1)
    wpe_il = wpe.astype(jnp.bfloat16).reshape(N_POS, DL2, 2, L).swapaxes(2, 3)
    wpe_pk = jax.lax.bitcast_convert_type(wpe_il, jnp.int32).reshape(
        N_POS, D // 2)
    hidden, mask = _emb_kernel(ids, pos, am, wte, wpe_pk)
    return (hidden.reshape(B, S, D), mask.reshape(1, 1, B, S))


# speedup vs baseline: 1.3785x; 1.3785x over previous
import functools

import jax
import jax.numpy as jnp
from jax import lax
from jax.experimental import pallas as pl
from jax.experimental.pallas import tpu as pltpu
from jax.experimental.pallas import tpu_sc as plsc

NC = 2
NS = 16
L = 16
NW = NC * NS

TOKENS = 8192
D = 768
R = TOKENS // NW
C = 64
NCH = R // C
DL = D // L

_mesh = plsc.VectorSubcoreMesh(core_axis_name="c", subcore_axis_name="s")


@functools.partial(
    pl.kernel,
    out_type=(
        jax.ShapeDtypeStruct((TOKENS, D), jnp.float32),
        jax.ShapeDtypeStruct((TOKENS,), jnp.float32),
    ),
    mesh=_mesh,
    scratch_types=[
        pltpu.VMEM((R,), jnp.int32),
        pltpu.VMEM((R,), jnp.int32),
        pltpu.VMEM((C, D), jnp.float32),
        pltpu.VMEM((C, D), jnp.float32),
        pltpu.VMEM((R,), jnp.float32),
        pltpu.SemaphoreType.DMA,
    ],
)
def _emb_kernel(ids_hbm, pos_hbm, am_hbm, wte_hbm, wpe_hbm, out_hbm, mask_hbm,
                tok_idx, pos_idx, tok_rows, pos_rows, am_v, sem):
    wid = lax.axis_index("s") * NC + lax.axis_index("c")
    base = wid * R

    pltpu.sync_copy(ids_hbm.at[pl.ds(base, R)], tok_idx)
    pltpu.sync_copy(pos_hbm.at[pl.ds(base, R)], pos_idx)

    @pl.loop(0, NCH)
    def _chunk(i):
        sl = pl.ds(i * C, C)
        h1 = pltpu.async_copy(wte_hbm.at[tok_idx.at[sl]], tok_rows, sem)
        h2 = pltpu.async_copy(wpe_hbm.at[pos_idx.at[sl]], pos_rows, sem)

        @pl.when(i == 0)
        def _():
            pltpu.sync_copy(am_hbm.at[pl.ds(base, R)], am_v)

            @pl.loop(0, R // L)
            def _mask(j):
                s = pl.ds(j * L, L)
                am_v[s] = (am_v[s] - 1.0) * 10000.0

            pltpu.sync_copy(am_v, mask_hbm.at[pl.ds(base, R)])

        h1.wait()
        h2.wait()

        @pl.loop(0, C)
        def _row(r):
            for j in range(DL):
                s = pl.ds(j * L, L)
                tok_rows[r, s] = tok_rows[r, s] + pos_rows[r, s]

        pltpu.sync_copy(tok_rows, out_hbm.at[pl.ds(base + i * C, C)])


def kernel(input_ids, attention_mask, position_ids, wte, wpe):
    B, S = input_ids.shape
    ids = input_ids.reshape(-1).astype(jnp.int32)
    pos = position_ids.reshape(-1).astype(jnp.int32)
    am = attention_mask.reshape(-1)
    hidden, mask = _emb_kernel(ids, pos, am, wte, wpe)
    return (hidden.reshape(B, S, D), mask.reshape(1, 1, B, S))
